# Initial kernel scaffold; baseline (speedup 1.0000x reference)
import functools

import jax
import jax.numpy as jnp
from jax import lax
from jax.experimental import pallas as pl
from jax.experimental.pallas import tpu as pltpu
from jax.experimental.pallas import tpu_sc as plsc

VOCAB = 1000000
EMBED = 32
NFEAT = 26
BATCH = 4096
HIDDEN = 512
NCLASS = 2

NC = 2
NS = 16
NW = NC * NS


def _gather_rows(emb, idx_flat):
    n = idx_flat.shape[0]
    bpw = n // NW
    mesh = plsc.VectorSubcoreMesh(core_axis_name="c", subcore_axis_name="s")

    @functools.partial(
        pl.kernel,
        out_type=jax.ShapeDtypeStruct((n, EMBED), jnp.float32),
        mesh=mesh,
        scratch_types=[
            pltpu.VMEM((bpw,), jnp.int32),
            pltpu.VMEM((bpw, EMBED), jnp.float32),
            pltpu.SemaphoreType.DMA,
        ],
    )
    def gather_kernel(emb_hbm, idx_hbm, out_hbm, idx_v, rows_v, sem):
        wid = lax.axis_index("s") * NC + lax.axis_index("c")
        base = wid * bpw
        pltpu.sync_copy(idx_hbm.at[pl.ds(base, bpw)], idx_v)
        pltpu.async_copy(emb_hbm.at[idx_v], rows_v, sem).wait()
        pltpu.sync_copy(rows_v, out_hbm.at[pl.ds(base, bpw)])

    return gather_kernel(emb, idx_flat)


def _mlp_block(x_ref, w1_ref, b1_ref, w2_ref, b2_ref, o_ref):
    h = jnp.dot(x_ref[...], w1_ref[...], preferred_element_type=jnp.float32)
    h = jnp.maximum(h + b1_ref[...], 0.0)
    o_ref[...] = (
        jnp.dot(h, w2_ref[...], preferred_element_type=jnp.float32) + b2_ref[...]
    )


def _mlp(e, W1, b1, W2, b2):
    bb = 512
    grid = (BATCH // bb,)
    return pl.pallas_call(
        _mlp_block,
        grid=grid,
        in_specs=[
            pl.BlockSpec((bb, NFEAT * EMBED), lambda i: (i, 0)),
            pl.BlockSpec((NFEAT * EMBED, HIDDEN), lambda i: (0, 0)),
            pl.BlockSpec((1, HIDDEN), lambda i: (0, 0)),
            pl.BlockSpec((HIDDEN, NCLASS), lambda i: (0, 0)),
            pl.BlockSpec((1, NCLASS), lambda i: (0, 0)),
        ],
        out_specs=pl.BlockSpec((bb, NCLASS), lambda i: (i, 0)),
        out_shape=jax.ShapeDtypeStruct((BATCH, NCLASS), jnp.float32),
    )(e, W1, b1.reshape(1, HIDDEN), W2, b2.reshape(1, NCLASS))


@jax.jit
def kernel(x, emb, W1, b1, W2, b2):
    idx = x.T.reshape(-1)
    rows = _gather_rows(emb, idx)
    e = rows.reshape(BATCH, NFEAT * EMBED)
    return _mlp(e, W1, b1, W2, b2)


# speedup vs baseline: 5.0732x; 5.0732x over previous
import functools

import jax
import jax.numpy as jnp
from jax import lax
from jax.experimental import pallas as pl
from jax.experimental.pallas import tpu as pltpu
from jax.experimental.pallas import tpu_sc as plsc

VOCAB = 1000000
EMBED = 32
NFEAT = 26
BATCH = 4096
HIDDEN = 512
NCLASS = 2

NC = 2
NS = 16
NW = NC * NS


def _gather_rows(emb, idx_flat):
    n = idx_flat.shape[0]
    bpw = n // NW
    mesh = plsc.VectorSubcoreMesh(core_axis_name="c", subcore_axis_name="s")

    @functools.partial(
        pl.kernel,
        out_type=jax.ShapeDtypeStruct((n, EMBED), jnp.float32),
        mesh=mesh,
        scratch_types=[
            pltpu.VMEM((bpw,), jnp.int32),
            pltpu.VMEM((bpw, EMBED), jnp.float32),
            pltpu.SemaphoreType.DMA,
        ],
        compiler_params=pltpu.CompilerParams(use_tc_tiling_on_sc=False),
    )
    def gather_kernel(emb_hbm, idx_hbm, out_hbm, idx_v, rows_v, sem):
        wid = lax.axis_index("s") * NC + lax.axis_index("c")
        base = wid * bpw
        pltpu.sync_copy(idx_hbm.at[pl.ds(base, bpw)], idx_v)
        pltpu.async_copy(emb_hbm.at[idx_v], rows_v, sem).wait()
        pltpu.sync_copy(rows_v, out_hbm.at[pl.ds(base, bpw)])

    return gather_kernel(emb, idx_flat)


def _mlp_block(x_ref, w1_ref, b1_ref, w2_ref, b2_ref, o_ref):
    h = jnp.dot(x_ref[...], w1_ref[...], preferred_element_type=jnp.float32)
    h = jnp.maximum(h + b1_ref[...], 0.0)
    o_ref[...] = (
        jnp.dot(h, w2_ref[...], preferred_element_type=jnp.float32) + b2_ref[...]
    )


def _mlp(e, W1, b1, W2, b2):
    bb = 512
    grid = (BATCH // bb,)
    return pl.pallas_call(
        _mlp_block,
        grid=grid,
        in_specs=[
            pl.BlockSpec((bb, NFEAT * EMBED), lambda i: (i, 0)),
            pl.BlockSpec((NFEAT * EMBED, HIDDEN), lambda i: (0, 0)),
            pl.BlockSpec((1, HIDDEN), lambda i: (0, 0)),
            pl.BlockSpec((HIDDEN, NCLASS), lambda i: (0, 0)),
            pl.BlockSpec((1, NCLASS), lambda i: (0, 0)),
        ],
        out_specs=pl.BlockSpec((bb, NCLASS), lambda i: (i, 0)),
        out_shape=jax.ShapeDtypeStruct((BATCH, NCLASS), jnp.float32),
    )(e, W1, b1.reshape(1, HIDDEN), W2, b2.reshape(1, NCLASS))


@jax.jit
def kernel(x, emb, W1, b1, W2, b2):
    idx = x.T.reshape(-1)
    rows = _gather_rows(emb, idx)
    e = rows.reshape(BATCH, NFEAT * EMBED)
    return _mlp(e, W1, b1, W2, b2)


# speedup vs baseline: 5.0775x; 1.0008x over previous
import functools

import jax
import jax.numpy as jnp
from jax import lax
from jax.experimental import pallas as pl
from jax.experimental.pallas import tpu as pltpu
from jax.experimental.pallas import tpu_sc as plsc

VOCAB = 1000000
EMBED = 32
NFEAT = 26
BATCH = 4096
HIDDEN = 512
NCLASS = 2

NC = 2
NS = 16
NW = NC * NS


def _gather_rows(emb, x):
    bpb = BATCH // NW
    mesh = plsc.VectorSubcoreMesh(core_axis_name="c", subcore_axis_name="s")

    @functools.partial(
        pl.kernel,
        out_type=jax.ShapeDtypeStruct((BATCH, NFEAT * EMBED), jnp.float32),
        mesh=mesh,
        scratch_types=[
            pltpu.VMEM((NFEAT, bpb), jnp.int32),
            pltpu.VMEM((NFEAT, bpb, EMBED), jnp.float32),
            pltpu.SemaphoreType.DMA,
            pltpu.SemaphoreType.DMA,
        ],
        compiler_params=pltpu.CompilerParams(use_tc_tiling_on_sc=False),
    )
    def gather_kernel(emb_hbm, x_hbm, out_hbm, xs_v, rows_v, gsem, wsem):
        wid = lax.axis_index("s") * NC + lax.axis_index("c")
        base = wid * bpb
        pltpu.sync_copy(x_hbm.at[:, pl.ds(base, bpb)], xs_v)

        def fire_gather(f, _):
            pltpu.async_copy(emb_hbm.at[xs_v.at[f]], rows_v.at[f], gsem)
            return ()

        lax.fori_loop(0, NFEAT, fire_gather, (), unroll=False)

        def drain_gather(f, _):
            pltpu.make_async_copy(emb_hbm.at[xs_v.at[f]], rows_v.at[f], gsem).wait()
            return ()

        lax.fori_loop(0, NFEAT, drain_gather, (), unroll=False)

        def fire_write(f, _):
            pltpu.async_copy(
                rows_v.at[f],
                out_hbm.at[pl.ds(base, bpb), pl.ds(f * EMBED, EMBED)],
                wsem,
            )
            return ()

        lax.fori_loop(0, NFEAT, fire_write, (), unroll=False)

        def drain_write(f, _):
            pltpu.make_async_copy(
                rows_v.at[f],
                out_hbm.at[pl.ds(base, bpb), pl.ds(f * EMBED, EMBED)],
                wsem,
            ).wait()
            return ()

        lax.fori_loop(0, NFEAT, drain_write, (), unroll=False)

    return gather_kernel(emb, x)


def _mlp_block(x_ref, w1_ref, b1_ref, w2_ref, b2_ref, o_ref):
    h = jnp.dot(x_ref[...], w1_ref[...], preferred_element_type=jnp.float32)
    h = jnp.maximum(h + b1_ref[...], 0.0)
    o_ref[...] = (
        jnp.dot(h, w2_ref[...], preferred_element_type=jnp.float32) + b2_ref[...]
    )


def _mlp(e, W1, b1, W2, b2):
    bb = 512
    grid = (BATCH // bb,)
    return pl.pallas_call(
        _mlp_block,
        grid=grid,
        in_specs=[
            pl.BlockSpec((bb, NFEAT * EMBED), lambda i: (i, 0)),
            pl.BlockSpec((NFEAT * EMBED, HIDDEN), lambda i: (0, 0)),
            pl.BlockSpec((1, HIDDEN), lambda i: (0, 0)),
            pl.BlockSpec((HIDDEN, NCLASS), lambda i: (0, 0)),
            pl.BlockSpec((1, NCLASS), lambda i: (0, 0)),
        ],
        out_specs=pl.BlockSpec((bb, NCLASS), lambda i: (i, 0)),
        out_shape=jax.ShapeDtypeStruct((BATCH, NCLASS), jnp.float32),
    )(e, W1, b1.reshape(1, HIDDEN), W2, b2.reshape(1, NCLASS))


@jax.jit
def kernel(x, emb, W1, b1, W2, b2):
    e = _gather_rows(emb, x)
    return _mlp(e, W1, b1, W2, b2)
